# baseline (speedup 1.0000x reference)
import functools

import jax
import jax.numpy as jnp
from jax import lax
from jax.experimental import pallas as pl
from jax.experimental.pallas import tpu as pltpu
from jax.experimental.pallas import tpu_sc as plsc

N = 10000
D = 128
E_MESH = 320000
E_WORLD = 80000
CH = 128
NTILES = 32
EPM = 327680
EPW = 98304
CPM = EPM // (NTILES * CH)
CPW = EPW // (NTILES * CH)
NACC = 10240
ZROWS = NACC // 16

_f32 = jnp.float32


def _proj_body(n_ref, w_ref, t_ref):
    t_ref[...] = jnp.dot(n_ref[...], w_ref[0], preferred_element_type=_f32)


def _project(node, ws):
    blk = 1000
    return pl.pallas_call(
        _proj_body,
        grid=(4, N // blk),
        in_specs=[
            pl.BlockSpec((blk, D), lambda j, i: (i, 0)),
            pl.BlockSpec((1, D, D), lambda j, i: (j, 0, 0)),
        ],
        out_specs=pl.BlockSpec((blk, D), lambda j, i: (j * (N // blk) + i, 0)),
        out_shape=jax.ShapeDtypeStruct((4 * N, D), _f32),
    )(node, ws)


@functools.cache
def _get_sc_gather():
    mesh = plsc.VectorSubcoreMesh(
        core_axis_name="c", subcore_axis_name="s",
        num_cores=2, num_subcores=16)
    return functools.partial(
        pl.kernel,
        out_type=[jax.ShapeDtypeStruct((EPM, D), _f32),
                  jax.ShapeDtypeStruct((EPM, D), _f32),
                  jax.ShapeDtypeStruct((EPW, D), _f32),
                  jax.ShapeDtypeStruct((EPW, D), _f32)],
        mesh=mesh,
        scratch_types=[
            pltpu.VMEM((CPM * CH,), jnp.int32),
            pltpu.VMEM((4, CH, D), _f32),
            pltpu.SemaphoreType.DMA,
            pltpu.SemaphoreType.DMA,
            pltpu.SemaphoreType.DMA,
            pltpu.SemaphoreType.DMA,
            pltpu.SemaphoreType.DMA,
            pltpu.SemaphoreType.DMA,
            pltpu.SemaphoreType.DMA,
            pltpu.SemaphoreType.DMA,
        ],
    )(_sc_gather_body)


def _sc_gather_body(t_hbm, ism_hbm, irm_hbm, isw_hbm, irw_hbm,
                    gsm_hbm, grm_hbm, gsw_hbm, grw_hbm,
                    idx_v, buf_v,
                    sg0, sg1, sg2, sg3, so0, so1, so2, so3):
    wid = lax.axis_index("s") * 2 + lax.axis_index("c")
    sgs = (sg0, sg1, sg2, sg3)
    sos = (so0, so1, so2, so3)
    LAG = 2

    def phase(nchunks, i_hbm, dst_hbm):
        n_idx = nchunks * CH
        pltpu.sync_copy(i_hbm.at[pl.ds(wid * n_idx, n_idx)],
                        idx_v.at[pl.ds(0, n_idx)])

        def issue(k, b):
            @pl.when(k >= 4)
            def _():
                pltpu.make_async_copy(
                    buf_v.at[b],
                    dst_hbm.at[pl.ds((wid * nchunks + k - 4) * CH, CH)],
                    sos[b]).wait()
            pltpu.async_copy(t_hbm.at[idx_v.at[pl.ds(k * CH, CH)]],
                             buf_v.at[b], sgs[b])

        def process(j, bp):
            pltpu.make_async_copy(
                t_hbm.at[idx_v.at[pl.ds(j * CH, CH)]],
                buf_v.at[bp], sgs[bp]).wait()
            pltpu.async_copy(
                buf_v.at[bp],
                dst_hbm.at[pl.ds((wid * nchunks + j) * CH, CH)], sos[bp])

        def outer(g, _):
            for b in range(4):
                k = 4 * g + b
                issue(k, b)
                j = k - LAG
                bp = (b - LAG) % 4

                @pl.when(j >= 0)
                def _():
                    process(j, bp)
            return 0

        lax.fori_loop(0, nchunks // 4, outer, 0)
        for j in (nchunks - 2, nchunks - 1):
            process(j, j % 4)
        for b in range(4):
            pltpu.make_async_copy(
                buf_v.at[b],
                dst_hbm.at[pl.ds((wid * nchunks + nchunks - 4 + b) * CH, CH)],
                sos[b]).wait()

    phase(CPM, ism_hbm, gsm_hbm)
    phase(CPM, irm_hbm, grm_hbm)
    phase(CPW, isw_hbm, gsw_hbm)
    phase(CPW, irw_hbm, grw_hbm)


def _edge_body(gs_ref, gr_ref, e_ref, w1c_ref, b1_ref, w2_ref, b2_ref,
               new_ref, out_ref):
    e = e_ref[...]
    pre = (gs_ref[...] + gr_ref[...]
           + jnp.dot(e, w1c_ref[...], preferred_element_type=_f32)
           + b1_ref[...])
    h = jnp.maximum(pre, 0.0)
    new = jnp.dot(h, w2_ref[...], preferred_element_type=_f32) + b2_ref[...]
    new_ref[...] = new
    out_ref[...] = new + e


def _edge_mlp(gs, gr, ef, w1c, b1, w2, b2, e_real):
    ep = gs.shape[0]
    blk = 2048
    grid = (e_real + blk - 1) // blk
    return pl.pallas_call(
        _edge_body,
        grid=(grid,),
        in_specs=[
            pl.BlockSpec((blk, D), lambda i: (i, 0)),
            pl.BlockSpec((blk, D), lambda i: (i, 0)),
            pl.BlockSpec((blk, D), lambda i: (i, 0)),
            pl.BlockSpec((D, D), lambda i: (0, 0)),
            pl.BlockSpec((1, D), lambda i: (0, 0)),
            pl.BlockSpec((D, D), lambda i: (0, 0)),
            pl.BlockSpec((1, D), lambda i: (0, 0)),
        ],
        out_specs=[
            pl.BlockSpec((blk, D), lambda i: (i, 0)),
            pl.BlockSpec((blk, D), lambda i: (i, 0)),
        ],
        out_shape=[jax.ShapeDtypeStruct((ep, D), _f32),
                   jax.ShapeDtypeStruct((e_real, D), _f32)],
    )(gs, gr, ef, w1c, b1, w2, b2)


@functools.cache
def _get_sc_scatter():
    mesh = plsc.VectorSubcoreMesh(
        core_axis_name="c", subcore_axis_name="s",
        num_cores=2, num_subcores=16)
    return functools.partial(
        pl.kernel,
        out_type=[jax.ShapeDtypeStruct((2, NACC, D), _f32),
                  jax.ShapeDtypeStruct((2, NACC, D), _f32)],
        mesh=mesh,
        scratch_types=[
            pltpu.VMEM((CH,), jnp.int32),
            pltpu.VMEM((CH, D), _f32),
            pltpu.VMEM_SHARED((NACC, D), _f32),
        ],
    )(_sc_scatter_body)


def _sc_scatter_body(nm_hbm, rm_hbm, nw_hbm, rw_hbm, z_hbm, am_hbm, aw_hbm,
                     i1_v, rows_v, acc):
    c = lax.axis_index("c")
    s = lax.axis_index("s")
    wid = s * 2 + c

    def phase(nchunks, r_hbm, src_hbm, out_hbm):
        pltpu.sync_copy(z_hbm, acc.at[pl.ds(s * ZROWS, ZROWS)])
        plsc.subcore_barrier()

        def body(k, _):
            base = (wid * nchunks + k) * CH
            pltpu.sync_copy(r_hbm.at[pl.ds(base, CH)], i1_v)
            pltpu.sync_copy(src_hbm.at[pl.ds(base, CH)], rows_v)
            pltpu.sync_copy(rows_v, acc.at[i1_v], add=True)
            return 0

        lax.fori_loop(0, nchunks, body, 0)
        plsc.subcore_barrier()
        pltpu.sync_copy(acc.at[pl.ds(s * ZROWS, ZROWS)],
                        out_hbm.at[c, pl.ds(s * ZROWS, ZROWS)])
        plsc.subcore_barrier()

    phase(CPM, rm_hbm, nm_hbm, am_hbm)
    phase(CPW, rw_hbm, nw_hbm, aw_hbm)


def _node_body(n_ref, am_ref, aw_ref, w_ref, b1_ref, w2_ref, b2_ref, o_ref):
    n = n_ref[...]
    am = am_ref[0] + am_ref[1]
    aw = aw_ref[0] + aw_ref[1]
    pre = (jnp.dot(n, w_ref[0], preferred_element_type=_f32)
           + jnp.dot(am, w_ref[1], preferred_element_type=_f32)
           + jnp.dot(aw, w_ref[2], preferred_element_type=_f32)
           + b1_ref[...])
    h = jnp.maximum(pre, 0.0)
    o_ref[...] = jnp.dot(h, w2_ref[...], preferred_element_type=_f32) \
        + b2_ref[...] + n


def _node_mlp(node, am_p, aw_p, nws, b1, w2, b2):
    blk = 1000
    return pl.pallas_call(
        _node_body,
        grid=(N // blk,),
        in_specs=[
            pl.BlockSpec((blk, D), lambda i: (i, 0)),
            pl.BlockSpec((2, blk, D), lambda i: (0, i, 0)),
            pl.BlockSpec((2, blk, D), lambda i: (0, i, 0)),
            pl.BlockSpec((3, D, D), lambda i: (0, 0, 0)),
            pl.BlockSpec((1, D), lambda i: (0, 0)),
            pl.BlockSpec((D, D), lambda i: (0, 0)),
            pl.BlockSpec((1, D), lambda i: (0, 0)),
        ],
        out_specs=pl.BlockSpec((blk, D), lambda i: (i, 0)),
        out_shape=jax.ShapeDtypeStruct((N, D), _f32),
    )(node, am_p, aw_p, nws, b1, w2, b2)


def kernel(node_features, mesh_edge_features, world_edge_features,
           mesh_senders, mesh_receivers, world_senders, world_receivers,
           mesh_W1, mesh_b1, mesh_W2, mesh_b2,
           world_W1, world_b1, world_W2, world_b2,
           node_W1, node_b1, node_W2, node_b2):
    pm = EPM - E_MESH
    pw = EPW - E_WORLD
    ism = jnp.pad(mesh_senders, (0, pm))
    irm = jnp.pad(mesh_receivers + N, (0, pm))
    isw = jnp.pad(world_senders + 2 * N, (0, pw))
    irw = jnp.pad(world_receivers + 3 * N, (0, pw))
    srm = jnp.pad(mesh_receivers, (0, pm), constant_values=N)
    srw = jnp.pad(world_receivers, (0, pw), constant_values=N)
    efm = jnp.pad(mesh_edge_features, ((0, pm), (0, 0)))
    efw = jnp.pad(world_edge_features, ((0, pw), (0, 0)))
    zeros = jnp.zeros((ZROWS, D), _f32)

    ws_proj = jnp.stack([mesh_W1[:D], mesh_W1[D:2 * D],
                         world_W1[:D], world_W1[D:2 * D]])
    nws = jnp.stack([node_W1[:D], node_W1[D:2 * D], node_W1[2 * D:]])

    t = _project(node_features, ws_proj)
    gsm, grm, gsw, grw = _get_sc_gather()(t, ism, irm, isw, irw)
    new_m, out_m = _edge_mlp(gsm, grm, efm, mesh_W1[2 * D:],
                             mesh_b1.reshape(1, D),
                             mesh_W2, mesh_b2.reshape(1, D), E_MESH)
    new_w, out_w = _edge_mlp(gsw, grw, efw, world_W1[2 * D:],
                             world_b1.reshape(1, D),
                             world_W2, world_b2.reshape(1, D), E_WORLD)
    am_p, aw_p = _get_sc_scatter()(new_m, srm, new_w, srw, zeros)
    out_n = _node_mlp(node_features, am_p, aw_p, nws,
                      node_b1.reshape(1, D), node_W2, node_b2.reshape(1, D))
    return (out_n, out_m, out_w)


# speedup vs baseline: 3.4265x; 3.4265x over previous
import functools

import jax
import jax.numpy as jnp
from jax import lax
from jax.experimental import pallas as pl
from jax.experimental.pallas import tpu as pltpu
from jax.experimental.pallas import tpu_sc as plsc

N = 10000
D = 128
E_MESH = 320000
E_WORLD = 80000
CH = 128
NTILES = 32
EPM = 327680
EPW = 98304
CPM = EPM // (NTILES * CH)
CPW = EPW // (NTILES * CH)
NACC = 10240
ZROWS = NACC // 16

_f32 = jnp.float32


def _proj_body(n_ref, w_ref, t_ref):
    t_ref[0] = jnp.dot(n_ref[...], w_ref[0], preferred_element_type=_f32)


def _project(node, ws):
    blk = 1000
    return pl.pallas_call(
        _proj_body,
        grid=(4, N // blk),
        in_specs=[
            pl.BlockSpec((blk, D), lambda j, i: (i, 0)),
            pl.BlockSpec((1, D, D), lambda j, i: (j, 0, 0)),
        ],
        out_specs=pl.BlockSpec((1, blk, D), lambda j, i: (j, i, 0)),
        out_shape=jax.ShapeDtypeStruct((4, NACC, D), _f32),
    )(node, ws)


@functools.cache
def _get_sc_gather():
    mesh = plsc.VectorSubcoreMesh(
        core_axis_name="c", subcore_axis_name="s",
        num_cores=2, num_subcores=16)
    return functools.partial(
        pl.kernel,
        out_type=[jax.ShapeDtypeStruct((EPM, D), _f32),
                  jax.ShapeDtypeStruct((EPM, D), _f32),
                  jax.ShapeDtypeStruct((EPW, D), _f32),
                  jax.ShapeDtypeStruct((EPW, D), _f32)],
        mesh=mesh,
        scratch_types=[
            pltpu.VMEM((CPM * CH,), jnp.int32),
            pltpu.VMEM((2, CH, D), _f32),
            pltpu.VMEM_SHARED((NACC, D), _f32),
            pltpu.SemaphoreType.DMA,
            pltpu.SemaphoreType.DMA,
            pltpu.SemaphoreType.DMA,
            pltpu.SemaphoreType.DMA,
        ],
    )(_sc_gather_body)


def _sc_gather_body(t_hbm, ism_hbm, irm_hbm, isw_hbm, irw_hbm,
                    gsm_hbm, grm_hbm, gsw_hbm, grw_hbm,
                    idx_v, buf_v, tab_sh,
                    sg0, sg1, so0, so1):
    wid = lax.axis_index("s") * 2 + lax.axis_index("c")
    s = lax.axis_index("s")
    sgs = (sg0, sg1)
    sos = (so0, so1)
    LAG = 1

    def phase(p, nchunks, i_hbm, dst_hbm):
        pltpu.sync_copy(t_hbm.at[p, pl.ds(s * ZROWS, ZROWS)],
                        tab_sh.at[pl.ds(s * ZROWS, ZROWS)])
        n_idx = nchunks * CH
        pltpu.sync_copy(i_hbm.at[pl.ds(wid * n_idx, n_idx)],
                        idx_v.at[pl.ds(0, n_idx)])
        plsc.subcore_barrier()

        def issue(k, b):
            @pl.when(k >= 2)
            def _():
                pltpu.make_async_copy(
                    buf_v.at[b],
                    dst_hbm.at[pl.ds((wid * nchunks + k - 2) * CH, CH)],
                    sos[b]).wait()
            pltpu.async_copy(tab_sh.at[idx_v.at[pl.ds(k * CH, CH)]],
                             buf_v.at[b], sgs[b])

        def process(j, bp):
            pltpu.make_async_copy(
                tab_sh.at[idx_v.at[pl.ds(j * CH, CH)]],
                buf_v.at[bp], sgs[bp]).wait()
            pltpu.async_copy(
                buf_v.at[bp],
                dst_hbm.at[pl.ds((wid * nchunks + j) * CH, CH)], sos[bp])

        def outer(g, _):
            for b in range(2):
                k = 2 * g + b
                issue(k, b)
                j = k - LAG
                bp = (b - LAG) % 2

                @pl.when(j >= 0)
                def _():
                    process(j, bp)
            return 0

        lax.fori_loop(0, nchunks // 2, outer, 0)
        process(nchunks - 1, (nchunks - 1) % 2)
        for b in range(2):
            pltpu.make_async_copy(
                buf_v.at[b],
                dst_hbm.at[pl.ds((wid * nchunks + nchunks - 2 + b) * CH, CH)],
                sos[b]).wait()
        plsc.subcore_barrier()

    phase(0, CPM, ism_hbm, gsm_hbm)
    phase(1, CPM, irm_hbm, grm_hbm)
    phase(2, CPW, isw_hbm, gsw_hbm)
    phase(3, CPW, irw_hbm, grw_hbm)


def _edge_body(gs_ref, gr_ref, e_ref, w1c_ref, b1_ref, w2_ref, b2_ref,
               new_ref, out_ref):
    e = e_ref[...]
    pre = (gs_ref[...] + gr_ref[...]
           + jnp.dot(e, w1c_ref[...], preferred_element_type=_f32)
           + b1_ref[...])
    h = jnp.maximum(pre, 0.0)
    new = jnp.dot(h, w2_ref[...], preferred_element_type=_f32) + b2_ref[...]
    new_ref[...] = new
    out_ref[...] = new + e


def _edge_mlp(gs, gr, ef, w1c, b1, w2, b2, e_real):
    ep = gs.shape[0]
    blk = 2048
    grid = (e_real + blk - 1) // blk
    return pl.pallas_call(
        _edge_body,
        grid=(grid,),
        in_specs=[
            pl.BlockSpec((blk, D), lambda i: (i, 0)),
            pl.BlockSpec((blk, D), lambda i: (i, 0)),
            pl.BlockSpec((blk, D), lambda i: (i, 0)),
            pl.BlockSpec((D, D), lambda i: (0, 0)),
            pl.BlockSpec((1, D), lambda i: (0, 0)),
            pl.BlockSpec((D, D), lambda i: (0, 0)),
            pl.BlockSpec((1, D), lambda i: (0, 0)),
        ],
        out_specs=[
            pl.BlockSpec((blk, D), lambda i: (i, 0)),
            pl.BlockSpec((blk, D), lambda i: (i, 0)),
        ],
        out_shape=[jax.ShapeDtypeStruct((ep, D), _f32),
                   jax.ShapeDtypeStruct((e_real, D), _f32)],
    )(gs, gr, ef, w1c, b1, w2, b2)


@functools.cache
def _get_sc_scatter():
    mesh = plsc.VectorSubcoreMesh(
        core_axis_name="c", subcore_axis_name="s",
        num_cores=2, num_subcores=16)
    return functools.partial(
        pl.kernel,
        out_type=[jax.ShapeDtypeStruct((2, NACC, D), _f32),
                  jax.ShapeDtypeStruct((2, NACC, D), _f32)],
        mesh=mesh,
        scratch_types=[
            pltpu.VMEM((CH,), jnp.int32),
            pltpu.VMEM((CH, D), _f32),
            pltpu.VMEM_SHARED((NACC, D), _f32),
        ],
    )(_sc_scatter_body)


def _sc_scatter_body(nm_hbm, rm_hbm, nw_hbm, rw_hbm, z_hbm, am_hbm, aw_hbm,
                     i1_v, rows_v, acc):
    c = lax.axis_index("c")
    s = lax.axis_index("s")
    wid = s * 2 + c

    def phase(nchunks, r_hbm, src_hbm, out_hbm):
        pltpu.sync_copy(z_hbm, acc.at[pl.ds(s * ZROWS, ZROWS)])
        plsc.subcore_barrier()

        def body(k, _):
            base = (wid * nchunks + k) * CH
            pltpu.sync_copy(r_hbm.at[pl.ds(base, CH)], i1_v)
            pltpu.sync_copy(src_hbm.at[pl.ds(base, CH)], rows_v)
            pltpu.sync_copy(rows_v, acc.at[i1_v], add=True)
            return 0

        lax.fori_loop(0, nchunks, body, 0)
        plsc.subcore_barrier()
        pltpu.sync_copy(acc.at[pl.ds(s * ZROWS, ZROWS)],
                        out_hbm.at[c, pl.ds(s * ZROWS, ZROWS)])
        plsc.subcore_barrier()

    phase(CPM, rm_hbm, nm_hbm, am_hbm)
    phase(CPW, rw_hbm, nw_hbm, aw_hbm)


def _node_body(n_ref, am_ref, aw_ref, w_ref, b1_ref, w2_ref, b2_ref, o_ref):
    n = n_ref[...]
    am = am_ref[0] + am_ref[1]
    aw = aw_ref[0] + aw_ref[1]
    pre = (jnp.dot(n, w_ref[0], preferred_element_type=_f32)
           + jnp.dot(am, w_ref[1], preferred_element_type=_f32)
           + jnp.dot(aw, w_ref[2], preferred_element_type=_f32)
           + b1_ref[...])
    h = jnp.maximum(pre, 0.0)
    o_ref[...] = jnp.dot(h, w2_ref[...], preferred_element_type=_f32) \
        + b2_ref[...] + n


def _node_mlp(node, am_p, aw_p, nws, b1, w2, b2):
    blk = 1000
    return pl.pallas_call(
        _node_body,
        grid=(N // blk,),
        in_specs=[
            pl.BlockSpec((blk, D), lambda i: (i, 0)),
            pl.BlockSpec((2, blk, D), lambda i: (0, i, 0)),
            pl.BlockSpec((2, blk, D), lambda i: (0, i, 0)),
            pl.BlockSpec((3, D, D), lambda i: (0, 0, 0)),
            pl.BlockSpec((1, D), lambda i: (0, 0)),
            pl.BlockSpec((D, D), lambda i: (0, 0)),
            pl.BlockSpec((1, D), lambda i: (0, 0)),
        ],
        out_specs=pl.BlockSpec((blk, D), lambda i: (i, 0)),
        out_shape=jax.ShapeDtypeStruct((N, D), _f32),
    )(node, am_p, aw_p, nws, b1, w2, b2)


def kernel(node_features, mesh_edge_features, world_edge_features,
           mesh_senders, mesh_receivers, world_senders, world_receivers,
           mesh_W1, mesh_b1, mesh_W2, mesh_b2,
           world_W1, world_b1, world_W2, world_b2,
           node_W1, node_b1, node_W2, node_b2):
    pm = EPM - E_MESH
    pw = EPW - E_WORLD
    gpad_m = jnp.arange(pm, dtype=jnp.int32) % N
    gpad_w = jnp.arange(pw, dtype=jnp.int32) % N
    ism = jnp.concatenate([mesh_senders, gpad_m])
    irm = jnp.concatenate([mesh_receivers, gpad_m])
    isw = jnp.concatenate([world_senders, gpad_w])
    irw = jnp.concatenate([world_receivers, gpad_w])
    spad_m = N + jnp.arange(pm, dtype=jnp.int32) % (NACC - N)
    spad_w = N + jnp.arange(pw, dtype=jnp.int32) % (NACC - N)
    srm = jnp.concatenate([mesh_receivers, spad_m])
    srw = jnp.concatenate([world_receivers, spad_w])
    efm = jnp.pad(mesh_edge_features, ((0, pm), (0, 0)))
    efw = jnp.pad(world_edge_features, ((0, pw), (0, 0)))
    zeros = jnp.zeros((ZROWS, D), _f32)

    ws_proj = jnp.stack([mesh_W1[:D], mesh_W1[D:2 * D],
                         world_W1[:D], world_W1[D:2 * D]])
    nws = jnp.stack([node_W1[:D], node_W1[D:2 * D], node_W1[2 * D:]])

    t = _project(node_features, ws_proj)
    gsm, grm, gsw, grw = _get_sc_gather()(t, ism, irm, isw, irw)
    new_m, out_m = _edge_mlp(gsm, grm, efm, mesh_W1[2 * D:],
                             mesh_b1.reshape(1, D),
                             mesh_W2, mesh_b2.reshape(1, D), E_MESH)
    new_w, out_w = _edge_mlp(gsw, grw, efw, world_W1[2 * D:],
                             world_b1.reshape(1, D),
                             world_W2, world_b2.reshape(1, D), E_WORLD)
    am_p, aw_p = _get_sc_scatter()(new_m, srm, new_w, srw, zeros)
    out_n = _node_mlp(node_features, am_p, aw_p, nws,
                      node_b1.reshape(1, D), node_W2, node_b2.reshape(1, D))
    return (out_n, out_m, out_w)


# speedup vs baseline: 3.9463x; 1.1517x over previous
import functools

import jax
import jax.numpy as jnp
from jax import lax
from jax.experimental import pallas as pl
from jax.experimental.pallas import tpu as pltpu
from jax.experimental.pallas import tpu_sc as plsc

N = 10000
D = 128
E_MESH = 320000
E_WORLD = 80000
CH = 128
NTILES = 32
EPM = 327680
EPW = 98304
CPM = EPM // (NTILES * CH)
CPW = EPW // (NTILES * CH)
NACC = 10240
ZROWS = NACC // 16

_f32 = jnp.float32


def _proj_body(n_ref, w_ref, t_ref):
    t_ref[0] = jnp.dot(n_ref[...], w_ref[0], preferred_element_type=_f32)


def _project(node, ws):
    blk = 1000
    return pl.pallas_call(
        _proj_body,
        grid=(4, N // blk),
        in_specs=[
            pl.BlockSpec((blk, D), lambda j, i: (i, 0)),
            pl.BlockSpec((1, D, D), lambda j, i: (j, 0, 0)),
        ],
        out_specs=pl.BlockSpec((1, blk, D), lambda j, i: (j, i, 0)),
        out_shape=jax.ShapeDtypeStruct((4, NACC, D), _f32),
    )(node, ws)


@functools.cache
def _get_sc_gather():
    mesh = plsc.VectorSubcoreMesh(
        core_axis_name="c", subcore_axis_name="s",
        num_cores=2, num_subcores=16)
    return functools.partial(
        pl.kernel,
        out_type=[jax.ShapeDtypeStruct((EPM, D), _f32),
                  jax.ShapeDtypeStruct((EPM, D), _f32),
                  jax.ShapeDtypeStruct((EPW, D), _f32),
                  jax.ShapeDtypeStruct((EPW, D), _f32)],
        mesh=mesh,
        scratch_types=[
            pltpu.VMEM((CPM * CH,), jnp.int32),
            pltpu.VMEM((2, CH, D), _f32),
            pltpu.VMEM_SHARED((NACC, D), _f32),
            pltpu.SemaphoreType.DMA,
            pltpu.SemaphoreType.DMA,
            pltpu.SemaphoreType.DMA,
            pltpu.SemaphoreType.DMA,
        ],
    )(_sc_gather_body)


def _sc_gather_body(t_hbm, ism_hbm, irm_hbm, isw_hbm, irw_hbm,
                    gsm_hbm, grm_hbm, gsw_hbm, grw_hbm,
                    idx_v, buf_v, tab_sh,
                    sg0, sg1, so0, so1):
    wid = lax.axis_index("s") * 2 + lax.axis_index("c")
    s = lax.axis_index("s")
    sgs = (sg0, sg1)
    sos = (so0, so1)
    LAG = 1

    def phase(p, nchunks, i_hbm, dst_hbm):
        pltpu.sync_copy(t_hbm.at[p, pl.ds(s * ZROWS, ZROWS)],
                        tab_sh.at[pl.ds(s * ZROWS, ZROWS)])
        n_idx = nchunks * CH
        pltpu.sync_copy(i_hbm.at[pl.ds(wid * n_idx, n_idx)],
                        idx_v.at[pl.ds(0, n_idx)])
        plsc.subcore_barrier()

        def issue(k, b):
            @pl.when(k >= 2)
            def _():
                pltpu.make_async_copy(
                    buf_v.at[b],
                    dst_hbm.at[pl.ds((wid * nchunks + k - 2) * CH, CH)],
                    sos[b]).wait()
            pltpu.async_copy(tab_sh.at[idx_v.at[pl.ds(k * CH, CH)]],
                             buf_v.at[b], sgs[b])

        def process(j, bp):
            pltpu.make_async_copy(
                tab_sh.at[idx_v.at[pl.ds(j * CH, CH)]],
                buf_v.at[bp], sgs[bp]).wait()
            pltpu.async_copy(
                buf_v.at[bp],
                dst_hbm.at[pl.ds((wid * nchunks + j) * CH, CH)], sos[bp])

        def outer(g, _):
            for b in range(2):
                k = 2 * g + b
                issue(k, b)
                j = k - LAG
                bp = (b - LAG) % 2

                @pl.when(j >= 0)
                def _():
                    process(j, bp)
            return 0

        lax.fori_loop(0, nchunks // 2, outer, 0)
        process(nchunks - 1, (nchunks - 1) % 2)
        for b in range(2):
            pltpu.make_async_copy(
                buf_v.at[b],
                dst_hbm.at[pl.ds((wid * nchunks + nchunks - 2 + b) * CH, CH)],
                sos[b]).wait()
        plsc.subcore_barrier()

    phase(0, CPM, ism_hbm, gsm_hbm)
    phase(1, CPM, irm_hbm, grm_hbm)
    phase(2, CPW, isw_hbm, gsw_hbm)
    phase(3, CPW, irw_hbm, grw_hbm)


def _edge_body(gs_ref, gr_ref, e_ref, w1c_ref, b1_ref, w2_ref, b2_ref,
               new_ref, out_ref):
    e = e_ref[...]
    pre = (gs_ref[...] + gr_ref[...]
           + jnp.dot(e, w1c_ref[...], preferred_element_type=_f32)
           + b1_ref[...])
    h = jnp.maximum(pre, 0.0)
    new = jnp.dot(h, w2_ref[...], preferred_element_type=_f32) + b2_ref[...]
    new_ref[...] = new
    out_ref[...] = new + e


def _edge_mlp(gs, gr, ef, w1c, b1, w2, b2, e_real):
    ep = gs.shape[0]
    blk = 2048
    grid = (e_real + blk - 1) // blk
    return pl.pallas_call(
        _edge_body,
        grid=(grid,),
        in_specs=[
            pl.BlockSpec((blk, D), lambda i: (i, 0)),
            pl.BlockSpec((blk, D), lambda i: (i, 0)),
            pl.BlockSpec((blk, D), lambda i: (i, 0)),
            pl.BlockSpec((D, D), lambda i: (0, 0)),
            pl.BlockSpec((1, D), lambda i: (0, 0)),
            pl.BlockSpec((D, D), lambda i: (0, 0)),
            pl.BlockSpec((1, D), lambda i: (0, 0)),
        ],
        out_specs=[
            pl.BlockSpec((blk, D), lambda i: (i, 0)),
            pl.BlockSpec((blk, D), lambda i: (i, 0)),
        ],
        out_shape=[jax.ShapeDtypeStruct((ep, D), _f32),
                   jax.ShapeDtypeStruct((e_real, D), _f32)],
    )(gs, gr, ef, w1c, b1, w2, b2)


@functools.cache
def _get_sc_scatter():
    mesh = plsc.VectorSubcoreMesh(
        core_axis_name="c", subcore_axis_name="s",
        num_cores=2, num_subcores=16)
    return functools.partial(
        pl.kernel,
        out_type=[jax.ShapeDtypeStruct((2, NACC, D), _f32),
                  jax.ShapeDtypeStruct((2, NACC, D), _f32)],
        mesh=mesh,
        scratch_types=[
            pltpu.VMEM((CPM * CH,), jnp.int32),
            pltpu.VMEM((2, CH, D), _f32),
            pltpu.VMEM_SHARED((NACC, D), _f32),
            pltpu.SemaphoreType.DMA,
            pltpu.SemaphoreType.DMA,
            pltpu.SemaphoreType.DMA,
            pltpu.SemaphoreType.DMA,
        ],
    )(_sc_scatter_body)


def _sc_scatter_body(nm_hbm, rm_hbm, nw_hbm, rw_hbm, z_hbm, am_hbm, aw_hbm,
                     idx_v, buf_v, acc, si0, si1, sa0, sa1):
    c = lax.axis_index("c")
    s = lax.axis_index("s")
    wid = s * 2 + c
    sis = (si0, si1)
    sas = (sa0, sa1)

    def phase(nchunks, r_hbm, src_hbm, out_hbm):
        pltpu.sync_copy(z_hbm, acc.at[pl.ds(s * ZROWS, ZROWS)])
        n_idx = nchunks * CH
        pltpu.sync_copy(r_hbm.at[pl.ds(wid * n_idx, n_idx)],
                        idx_v.at[pl.ds(0, n_idx)])
        plsc.subcore_barrier()

        def issue(k, b):
            @pl.when(k >= 2)
            def _():
                pltpu.make_async_copy(
                    buf_v.at[b],
                    acc.at[idx_v.at[pl.ds((k - 2) * CH, CH)]],
                    sas[b]).wait()
            pltpu.async_copy(
                src_hbm.at[pl.ds((wid * nchunks + k) * CH, CH)],
                buf_v.at[b], sis[b])

        def process(j, bp):
            pltpu.make_async_copy(
                src_hbm.at[pl.ds((wid * nchunks + j) * CH, CH)],
                buf_v.at[bp], sis[bp]).wait()
            pltpu.async_copy(buf_v.at[bp],
                             acc.at[idx_v.at[pl.ds(j * CH, CH)]],
                             sas[bp], add=True)

        def outer(g, _):
            for b in range(2):
                k = 2 * g + b
                issue(k, b)
                j = k - 1
                bp = (b - 1) % 2

                @pl.when(j >= 0)
                def _():
                    process(j, bp)
            return 0

        lax.fori_loop(0, nchunks // 2, outer, 0)
        process(nchunks - 1, (nchunks - 1) % 2)
        for b in range(2):
            pltpu.make_async_copy(
                buf_v.at[b],
                acc.at[idx_v.at[pl.ds((nchunks - 2 + b) * CH, CH)]],
                sas[b]).wait()
        plsc.subcore_barrier()
        pltpu.sync_copy(acc.at[pl.ds(s * ZROWS, ZROWS)],
                        out_hbm.at[c, pl.ds(s * ZROWS, ZROWS)])
        plsc.subcore_barrier()

    phase(CPM, rm_hbm, nm_hbm, am_hbm)
    phase(CPW, rw_hbm, nw_hbm, aw_hbm)


def _node_body(n_ref, am_ref, aw_ref, w_ref, b1_ref, w2_ref, b2_ref, o_ref):
    n = n_ref[...]
    am = am_ref[0] + am_ref[1]
    aw = aw_ref[0] + aw_ref[1]
    pre = (jnp.dot(n, w_ref[0], preferred_element_type=_f32)
           + jnp.dot(am, w_ref[1], preferred_element_type=_f32)
           + jnp.dot(aw, w_ref[2], preferred_element_type=_f32)
           + b1_ref[...])
    h = jnp.maximum(pre, 0.0)
    o_ref[...] = jnp.dot(h, w2_ref[...], preferred_element_type=_f32) \
        + b2_ref[...] + n


def _node_mlp(node, am_p, aw_p, nws, b1, w2, b2):
    blk = 1000
    return pl.pallas_call(
        _node_body,
        grid=(N // blk,),
        in_specs=[
            pl.BlockSpec((blk, D), lambda i: (i, 0)),
            pl.BlockSpec((2, blk, D), lambda i: (0, i, 0)),
            pl.BlockSpec((2, blk, D), lambda i: (0, i, 0)),
            pl.BlockSpec((3, D, D), lambda i: (0, 0, 0)),
            pl.BlockSpec((1, D), lambda i: (0, 0)),
            pl.BlockSpec((D, D), lambda i: (0, 0)),
            pl.BlockSpec((1, D), lambda i: (0, 0)),
        ],
        out_specs=pl.BlockSpec((blk, D), lambda i: (i, 0)),
        out_shape=jax.ShapeDtypeStruct((N, D), _f32),
    )(node, am_p, aw_p, nws, b1, w2, b2)


def kernel(node_features, mesh_edge_features, world_edge_features,
           mesh_senders, mesh_receivers, world_senders, world_receivers,
           mesh_W1, mesh_b1, mesh_W2, mesh_b2,
           world_W1, world_b1, world_W2, world_b2,
           node_W1, node_b1, node_W2, node_b2):
    pm = EPM - E_MESH
    pw = EPW - E_WORLD
    gpad_m = jnp.arange(pm, dtype=jnp.int32) % N
    gpad_w = jnp.arange(pw, dtype=jnp.int32) % N
    ism = jnp.concatenate([mesh_senders, gpad_m])
    irm = jnp.concatenate([mesh_receivers, gpad_m])
    isw = jnp.concatenate([world_senders, gpad_w])
    irw = jnp.concatenate([world_receivers, gpad_w])
    spad_m = N + jnp.arange(pm, dtype=jnp.int32) % (NACC - N)
    spad_w = N + jnp.arange(pw, dtype=jnp.int32) % (NACC - N)
    srm = jnp.concatenate([mesh_receivers, spad_m])
    srw = jnp.concatenate([world_receivers, spad_w])
    efm = jnp.pad(mesh_edge_features, ((0, pm), (0, 0)))
    efw = jnp.pad(world_edge_features, ((0, pw), (0, 0)))
    zeros = jnp.zeros((ZROWS, D), _f32)

    ws_proj = jnp.stack([mesh_W1[:D], mesh_W1[D:2 * D],
                         world_W1[:D], world_W1[D:2 * D]])
    nws = jnp.stack([node_W1[:D], node_W1[D:2 * D], node_W1[2 * D:]])

    t = _project(node_features, ws_proj)
    gsm, grm, gsw, grw = _get_sc_gather()(t, ism, irm, isw, irw)
    new_m, out_m = _edge_mlp(gsm, grm, efm, mesh_W1[2 * D:],
                             mesh_b1.reshape(1, D),
                             mesh_W2, mesh_b2.reshape(1, D), E_MESH)
    new_w, out_w = _edge_mlp(gsw, grw, efw, world_W1[2 * D:],
                             world_b1.reshape(1, D),
                             world_W2, world_b2.reshape(1, D), E_WORLD)
    am_p, aw_p = _get_sc_scatter()(new_m, srm, new_w, srw, zeros)
    out_n = _node_mlp(node_features, am_p, aw_p, nws,
                      node_b1.reshape(1, D), node_W2, node_b2.reshape(1, D))
    return (out_n, out_m, out_w)


# speedup vs baseline: 4.1676x; 1.0561x over previous
import functools

import jax
import jax.numpy as jnp
from jax import lax
from jax.experimental import pallas as pl
from jax.experimental.pallas import tpu as pltpu
from jax.experimental.pallas import tpu_sc as plsc

N = 10000
D = 128
E_MESH = 320000
E_WORLD = 80000
CH = 128
NTILES = 32
EPM = 327680
EPW = 98304
CPM = EPM // (NTILES * CH)
CPW = EPW // (NTILES * CH)
NACC = 10240
ZROWS = NACC // 16

_f32 = jnp.float32


def _proj_body(n_ref, w_ref, t_ref):
    t_ref[0] = jnp.dot(n_ref[...], w_ref[0], preferred_element_type=_f32)


def _project(node, ws):
    blk = 1000
    return pl.pallas_call(
        _proj_body,
        grid=(4, N // blk),
        in_specs=[
            pl.BlockSpec((blk, D), lambda j, i: (i, 0)),
            pl.BlockSpec((1, D, D), lambda j, i: (j, 0, 0)),
        ],
        out_specs=pl.BlockSpec((1, blk, D), lambda j, i: (j, i, 0)),
        out_shape=jax.ShapeDtypeStruct((4, NACC, D), _f32),
    )(node, ws)


@functools.cache
def _get_sc_gather(nchunks, ep, p_base):
    mesh = plsc.VectorSubcoreMesh(
        core_axis_name="c", subcore_axis_name="s",
        num_cores=2, num_subcores=16)

    def body(t_hbm, is_hbm, ir_hbm, gs_hbm, gr_hbm,
             idx_v, buf_v, tab_sh, sg0, sg1, so0, so1):
        wid = lax.axis_index("s") * 2 + lax.axis_index("c")
        s = lax.axis_index("s")
        sgs = (sg0, sg1)
        sos = (so0, so1)

        def phase(p, i_hbm, dst_hbm):
            pltpu.sync_copy(t_hbm.at[p, pl.ds(s * ZROWS, ZROWS)],
                            tab_sh.at[pl.ds(s * ZROWS, ZROWS)])
            n_idx = nchunks * CH
            pltpu.sync_copy(i_hbm.at[pl.ds(wid * n_idx, n_idx)], idx_v)
            plsc.subcore_barrier()

            def issue(k, b):
                @pl.when(k >= 2)
                def _():
                    pltpu.make_async_copy(
                        buf_v.at[b],
                        dst_hbm.at[pl.ds((wid * nchunks + k - 2) * CH, CH)],
                        sos[b]).wait()
                pltpu.async_copy(tab_sh.at[idx_v.at[pl.ds(k * CH, CH)]],
                                 buf_v.at[b], sgs[b])

            def process(j, bp):
                pltpu.make_async_copy(
                    tab_sh.at[idx_v.at[pl.ds(j * CH, CH)]],
                    buf_v.at[bp], sgs[bp]).wait()
                pltpu.async_copy(
                    buf_v.at[bp],
                    dst_hbm.at[pl.ds((wid * nchunks + j) * CH, CH)], sos[bp])

            def outer(g, _):
                for b in range(2):
                    k = 2 * g + b
                    issue(k, b)
                    j = k - 1
                    bp = (b - 1) % 2

                    @pl.when(j >= 0)
                    def _():
                        process(j, bp)
                return 0

            lax.fori_loop(0, nchunks // 2, outer, 0)
            process(nchunks - 1, (nchunks - 1) % 2)
            for b in range(2):
                pltpu.make_async_copy(
                    buf_v.at[b],
                    dst_hbm.at[
                        pl.ds((wid * nchunks + nchunks - 2 + b) * CH, CH)],
                    sos[b]).wait()
            plsc.subcore_barrier()

        phase(p_base, is_hbm, gs_hbm)
        phase(p_base + 1, ir_hbm, gr_hbm)

    return functools.partial(
        pl.kernel,
        out_type=[jax.ShapeDtypeStruct((ep, D), _f32),
                  jax.ShapeDtypeStruct((ep, D), _f32)],
        mesh=mesh,
        scratch_types=[
            pltpu.VMEM((nchunks * CH,), jnp.int32),
            pltpu.VMEM((2, CH, D), _f32),
            pltpu.VMEM_SHARED((NACC, D), _f32),
            pltpu.SemaphoreType.DMA,
            pltpu.SemaphoreType.DMA,
            pltpu.SemaphoreType.DMA,
            pltpu.SemaphoreType.DMA,
        ],
    )(body)


def _edge_body(gs_ref, gr_ref, e_ref, w1c_ref, b1_ref, w2_ref, b2_ref,
               new_ref, out_ref):
    e = e_ref[...]
    pre = (gs_ref[...] + gr_ref[...]
           + jnp.dot(e, w1c_ref[...], preferred_element_type=_f32)
           + b1_ref[...])
    h = jnp.maximum(pre, 0.0)
    new = jnp.dot(h, w2_ref[...], preferred_element_type=_f32) + b2_ref[...]
    new_ref[...] = new
    out_ref[...] = new + e


def _edge_mlp(gs, gr, ef, w1c, b1, w2, b2, e_real):
    ep = gs.shape[0]
    blk = 2048
    grid = (e_real + blk - 1) // blk
    return pl.pallas_call(
        _edge_body,
        grid=(grid,),
        in_specs=[
            pl.BlockSpec((blk, D), lambda i: (i, 0)),
            pl.BlockSpec((blk, D), lambda i: (i, 0)),
            pl.BlockSpec((blk, D), lambda i: (i, 0)),
            pl.BlockSpec((D, D), lambda i: (0, 0)),
            pl.BlockSpec((1, D), lambda i: (0, 0)),
            pl.BlockSpec((D, D), lambda i: (0, 0)),
            pl.BlockSpec((1, D), lambda i: (0, 0)),
        ],
        out_specs=[
            pl.BlockSpec((blk, D), lambda i: (i, 0)),
            pl.BlockSpec((blk, D), lambda i: (i, 0)),
        ],
        out_shape=[jax.ShapeDtypeStruct((ep, D), _f32),
                   jax.ShapeDtypeStruct((e_real, D), _f32)],
    )(gs, gr, ef, w1c, b1, w2, b2)


@functools.cache
def _get_sc_scatter(nchunks):
    mesh = plsc.VectorSubcoreMesh(
        core_axis_name="c", subcore_axis_name="s",
        num_cores=2, num_subcores=16)

    def body(src_hbm, r_hbm, z_hbm, out_hbm, idx_v, buf_v, acc,
             si0, si1, sa0, sa1):
        c = lax.axis_index("c")
        s = lax.axis_index("s")
        wid = s * 2 + c
        sis = (si0, si1)
        sas = (sa0, sa1)

        pltpu.sync_copy(z_hbm, acc.at[pl.ds(s * ZROWS, ZROWS)])
        n_idx = nchunks * CH
        pltpu.sync_copy(r_hbm.at[pl.ds(wid * n_idx, n_idx)], idx_v)
        plsc.subcore_barrier()

        def issue(k, b):
            @pl.when(k >= 2)
            def _():
                pltpu.make_async_copy(
                    buf_v.at[b],
                    acc.at[idx_v.at[pl.ds((k - 2) * CH, CH)]],
                    sas[b]).wait()
            pltpu.async_copy(
                src_hbm.at[pl.ds((wid * nchunks + k) * CH, CH)],
                buf_v.at[b], sis[b])

        def process(j, bp):
            pltpu.make_async_copy(
                src_hbm.at[pl.ds((wid * nchunks + j) * CH, CH)],
                buf_v.at[bp], sis[bp]).wait()
            pltpu.async_copy(buf_v.at[bp],
                             acc.at[idx_v.at[pl.ds(j * CH, CH)]],
                             sas[bp], add=True)

        def outer(g, _):
            for b in range(2):
                k = 2 * g + b
                issue(k, b)
                j = k - 1
                bp = (b - 1) % 2

                @pl.when(j >= 0)
                def _():
                    process(j, bp)
            return 0

        lax.fori_loop(0, nchunks // 2, outer, 0)
        process(nchunks - 1, (nchunks - 1) % 2)
        for b in range(2):
            pltpu.make_async_copy(
                buf_v.at[b],
                acc.at[idx_v.at[pl.ds((nchunks - 2 + b) * CH, CH)]],
                sas[b]).wait()
        plsc.subcore_barrier()
        pltpu.sync_copy(acc.at[pl.ds(s * ZROWS, ZROWS)],
                        out_hbm.at[c, pl.ds(s * ZROWS, ZROWS)])

    return functools.partial(
        pl.kernel,
        out_type=jax.ShapeDtypeStruct((2, NACC, D), _f32),
        mesh=mesh,
        scratch_types=[
            pltpu.VMEM((nchunks * CH,), jnp.int32),
            pltpu.VMEM((2, CH, D), _f32),
            pltpu.VMEM_SHARED((NACC, D), _f32),
            pltpu.SemaphoreType.DMA,
            pltpu.SemaphoreType.DMA,
            pltpu.SemaphoreType.DMA,
            pltpu.SemaphoreType.DMA,
        ],
    )(body)


def _node_body(n_ref, am_ref, aw_ref, w_ref, b1_ref, w2_ref, b2_ref, o_ref):
    n = n_ref[...]
    am = am_ref[0] + am_ref[1]
    aw = aw_ref[0] + aw_ref[1]
    pre = (jnp.dot(n, w_ref[0], preferred_element_type=_f32)
           + jnp.dot(am, w_ref[1], preferred_element_type=_f32)
           + jnp.dot(aw, w_ref[2], preferred_element_type=_f32)
           + b1_ref[...])
    h = jnp.maximum(pre, 0.0)
    o_ref[...] = jnp.dot(h, w2_ref[...], preferred_element_type=_f32) \
        + b2_ref[...] + n


def _node_mlp(node, am_p, aw_p, nws, b1, w2, b2):
    blk = 1000
    return pl.pallas_call(
        _node_body,
        grid=(N // blk,),
        in_specs=[
            pl.BlockSpec((blk, D), lambda i: (i, 0)),
            pl.BlockSpec((2, blk, D), lambda i: (0, i, 0)),
            pl.BlockSpec((2, blk, D), lambda i: (0, i, 0)),
            pl.BlockSpec((3, D, D), lambda i: (0, 0, 0)),
            pl.BlockSpec((1, D), lambda i: (0, 0)),
            pl.BlockSpec((D, D), lambda i: (0, 0)),
            pl.BlockSpec((1, D), lambda i: (0, 0)),
        ],
        out_specs=pl.BlockSpec((blk, D), lambda i: (i, 0)),
        out_shape=jax.ShapeDtypeStruct((N, D), _f32),
    )(node, am_p, aw_p, nws, b1, w2, b2)


def kernel(node_features, mesh_edge_features, world_edge_features,
           mesh_senders, mesh_receivers, world_senders, world_receivers,
           mesh_W1, mesh_b1, mesh_W2, mesh_b2,
           world_W1, world_b1, world_W2, world_b2,
           node_W1, node_b1, node_W2, node_b2):
    pm = EPM - E_MESH
    pw = EPW - E_WORLD
    gpad_m = jnp.arange(pm, dtype=jnp.int32) % N
    gpad_w = jnp.arange(pw, dtype=jnp.int32) % N
    ism = jnp.concatenate([mesh_senders, gpad_m])
    irm = jnp.concatenate([mesh_receivers, gpad_m])
    isw = jnp.concatenate([world_senders, gpad_w])
    irw = jnp.concatenate([world_receivers, gpad_w])
    spad_m = N + jnp.arange(pm, dtype=jnp.int32) % (NACC - N)
    spad_w = N + jnp.arange(pw, dtype=jnp.int32) % (NACC - N)
    srm = jnp.concatenate([mesh_receivers, spad_m])
    srw = jnp.concatenate([world_receivers, spad_w])
    efm = jnp.pad(mesh_edge_features, ((0, pm), (0, 0)))
    efw = jnp.pad(world_edge_features, ((0, pw), (0, 0)))
    zeros = jnp.zeros((ZROWS, D), _f32)

    ws_proj = jnp.stack([mesh_W1[:D], mesh_W1[D:2 * D],
                         world_W1[:D], world_W1[D:2 * D]])
    nws = jnp.stack([node_W1[:D], node_W1[D:2 * D], node_W1[2 * D:]])

    t = _project(node_features, ws_proj)
    gsm, grm = _get_sc_gather(CPM, EPM, 0)(t, ism, irm)
    gsw, grw = _get_sc_gather(CPW, EPW, 2)(t, isw, irw)
    new_m, out_m = _edge_mlp(gsm, grm, efm, mesh_W1[2 * D:],
                             mesh_b1.reshape(1, D),
                             mesh_W2, mesh_b2.reshape(1, D), E_MESH)
    am_p = _get_sc_scatter(CPM)(new_m, srm, zeros)
    new_w, out_w = _edge_mlp(gsw, grw, efw, world_W1[2 * D:],
                             world_b1.reshape(1, D),
                             world_W2, world_b2.reshape(1, D), E_WORLD)
    aw_p = _get_sc_scatter(CPW)(new_w, srw, zeros)
    out_n = _node_mlp(node_features, am_p, aw_p, nws,
                      node_b1.reshape(1, D), node_W2, node_b2.reshape(1, D))
    return (out_n, out_m, out_w)


# speedup vs baseline: 4.1756x; 1.0019x over previous
import functools

import jax
import jax.numpy as jnp
from jax import lax
from jax.experimental import pallas as pl
from jax.experimental.pallas import tpu as pltpu
from jax.experimental.pallas import tpu_sc as plsc

N = 10000
D = 128
E_MESH = 320000
E_WORLD = 80000
CH = 128
NTILES = 32
EPM = 327680
EPW = 98304
CPM = EPM // (NTILES * CH)
CPW = EPW // (NTILES * CH)
NACC = 10240
ZROWS = NACC // 16

_f32 = jnp.float32


def _proj_body(n_ref, w_ref, t_ref):
    p = jnp.dot(n_ref[...], w_ref[0], preferred_element_type=_f32)
    u = jax.lax.bitcast_convert_type(p, jnp.uint32)
    r = (u + jnp.uint32(0x7FFF) + ((u >> 16) & jnp.uint32(1))) >> 16
    a = r[:, :D // 2]
    b = r[:, D // 2:]
    t_ref[0] = jax.lax.bitcast_convert_type(a | (b << 16), jnp.int32)


def _project(node, ws):
    blk = 1000
    return pl.pallas_call(
        _proj_body,
        grid=(4, N // blk),
        in_specs=[
            pl.BlockSpec((blk, D), lambda j, i: (i, 0)),
            pl.BlockSpec((1, D, D), lambda j, i: (j, 0, 0)),
        ],
        out_specs=pl.BlockSpec((1, blk, D // 2), lambda j, i: (j, i, 0)),
        out_shape=jax.ShapeDtypeStruct((4, NACC, D // 2), jnp.int32),
    )(node, ws)


@functools.cache
def _get_sc_gather(nchunks, ep, p_base):
    mesh = plsc.VectorSubcoreMesh(
        core_axis_name="c", subcore_axis_name="s",
        num_cores=2, num_subcores=16)

    def body(t_hbm, is_hbm, ir_hbm, gs_hbm, gr_hbm,
             idx_v, buf_v, tab_sh, sg0, sg1, so0, so1):
        wid = lax.axis_index("s") * 2 + lax.axis_index("c")
        s = lax.axis_index("s")
        sgs = (sg0, sg1)
        sos = (so0, so1)

        def phase(p, i_hbm, dst_hbm):
            pltpu.sync_copy(t_hbm.at[p, pl.ds(s * ZROWS, ZROWS)],
                            tab_sh.at[pl.ds(s * ZROWS, ZROWS)])
            n_idx = nchunks * CH
            pltpu.sync_copy(i_hbm.at[pl.ds(wid * n_idx, n_idx)], idx_v)
            plsc.subcore_barrier()

            def issue(k, b):
                @pl.when(k >= 2)
                def _():
                    pltpu.make_async_copy(
                        buf_v.at[b],
                        dst_hbm.at[pl.ds((wid * nchunks + k - 2) * CH, CH)],
                        sos[b]).wait()
                pltpu.async_copy(tab_sh.at[idx_v.at[pl.ds(k * CH, CH)]],
                                 buf_v.at[b], sgs[b])

            def process(j, bp):
                pltpu.make_async_copy(
                    tab_sh.at[idx_v.at[pl.ds(j * CH, CH)]],
                    buf_v.at[bp], sgs[bp]).wait()
                pltpu.async_copy(
                    buf_v.at[bp],
                    dst_hbm.at[pl.ds((wid * nchunks + j) * CH, CH)], sos[bp])

            def outer(g, _):
                for b in range(2):
                    k = 2 * g + b
                    issue(k, b)
                    j = k - 1
                    bp = (b - 1) % 2

                    @pl.when(j >= 0)
                    def _():
                        process(j, bp)
                return 0

            lax.fori_loop(0, nchunks // 2, outer, 0)
            process(nchunks - 1, (nchunks - 1) % 2)
            for b in range(2):
                pltpu.make_async_copy(
                    buf_v.at[b],
                    dst_hbm.at[
                        pl.ds((wid * nchunks + nchunks - 2 + b) * CH, CH)],
                    sos[b]).wait()
            plsc.subcore_barrier()

        phase(p_base, is_hbm, gs_hbm)
        phase(p_base + 1, ir_hbm, gr_hbm)

    return functools.partial(
        pl.kernel,
        out_type=[jax.ShapeDtypeStruct((ep, D // 2), jnp.int32),
                  jax.ShapeDtypeStruct((ep, D // 2), jnp.int32)],
        mesh=mesh,
        scratch_types=[
            pltpu.VMEM((nchunks * CH,), jnp.int32),
            pltpu.VMEM((2, CH, D // 2), jnp.int32),
            pltpu.VMEM_SHARED((NACC, D // 2), jnp.int32),
            pltpu.SemaphoreType.DMA,
            pltpu.SemaphoreType.DMA,
            pltpu.SemaphoreType.DMA,
            pltpu.SemaphoreType.DMA,
        ],
    )(body)


def _unpack(g):
    u = jax.lax.bitcast_convert_type(g, jnp.uint32)
    lo = jax.lax.bitcast_convert_type(u << 16, _f32)
    hi = jax.lax.bitcast_convert_type(u & jnp.uint32(0xFFFF0000), _f32)
    return jnp.concatenate([lo, hi], axis=1)


def _edge_body(gs_ref, gr_ref, e_ref, w1c_ref, b1_ref, w2_ref, b2_ref,
               new_ref, out_ref):
    e = e_ref[...]
    pre = (_unpack(gs_ref[...]) + _unpack(gr_ref[...])
           + jnp.dot(e, w1c_ref[...], preferred_element_type=_f32)
           + b1_ref[...])
    h = jnp.maximum(pre, 0.0)
    new = jnp.dot(h, w2_ref[...], preferred_element_type=_f32) + b2_ref[...]
    new_ref[...] = new
    out_ref[...] = new + e


def _edge_mlp(gs, gr, ef, w1c, b1, w2, b2, e_real):
    ep = gs.shape[0]
    blk = 2048
    grid = (e_real + blk - 1) // blk
    return pl.pallas_call(
        _edge_body,
        grid=(grid,),
        in_specs=[
            pl.BlockSpec((blk, D // 2), lambda i: (i, 0)),
            pl.BlockSpec((blk, D // 2), lambda i: (i, 0)),
            pl.BlockSpec((blk, D), lambda i: (i, 0)),
            pl.BlockSpec((D, D), lambda i: (0, 0)),
            pl.BlockSpec((1, D), lambda i: (0, 0)),
            pl.BlockSpec((D, D), lambda i: (0, 0)),
            pl.BlockSpec((1, D), lambda i: (0, 0)),
        ],
        out_specs=[
            pl.BlockSpec((blk, D), lambda i: (i, 0)),
            pl.BlockSpec((blk, D), lambda i: (i, 0)),
        ],
        out_shape=[jax.ShapeDtypeStruct((ep, D), _f32),
                   jax.ShapeDtypeStruct((e_real, D), _f32)],
    )(gs, gr, ef, w1c, b1, w2, b2)


@functools.cache
def _get_sc_scatter(nchunks):
    mesh = plsc.VectorSubcoreMesh(
        core_axis_name="c", subcore_axis_name="s",
        num_cores=2, num_subcores=16)

    def body(src_hbm, r_hbm, z_hbm, out_hbm, idx_v, buf_v, acc,
             si0, si1, sa0, sa1):
        c = lax.axis_index("c")
        s = lax.axis_index("s")
        wid = s * 2 + c
        sis = (si0, si1)
        sas = (sa0, sa1)

        pltpu.sync_copy(z_hbm, acc.at[pl.ds(s * ZROWS, ZROWS)])
        n_idx = nchunks * CH
        pltpu.sync_copy(r_hbm.at[pl.ds(wid * n_idx, n_idx)], idx_v)
        plsc.subcore_barrier()

        def issue(k, b):
            @pl.when(k >= 2)
            def _():
                pltpu.make_async_copy(
                    buf_v.at[b],
                    acc.at[idx_v.at[pl.ds((k - 2) * CH, CH)]],
                    sas[b]).wait()
            pltpu.async_copy(
                src_hbm.at[pl.ds((wid * nchunks + k) * CH, CH)],
                buf_v.at[b], sis[b])

        def process(j, bp):
            pltpu.make_async_copy(
                src_hbm.at[pl.ds((wid * nchunks + j) * CH, CH)],
                buf_v.at[bp], sis[bp]).wait()
            pltpu.async_copy(buf_v.at[bp],
                             acc.at[idx_v.at[pl.ds(j * CH, CH)]],
                             sas[bp], add=True)

        def outer(g, _):
            for b in range(2):
                k = 2 * g + b
                issue(k, b)
                j = k - 1
                bp = (b - 1) % 2

                @pl.when(j >= 0)
                def _():
                    process(j, bp)
            return 0

        lax.fori_loop(0, nchunks // 2, outer, 0)
        process(nchunks - 1, (nchunks - 1) % 2)
        for b in range(2):
            pltpu.make_async_copy(
                buf_v.at[b],
                acc.at[idx_v.at[pl.ds((nchunks - 2 + b) * CH, CH)]],
                sas[b]).wait()
        plsc.subcore_barrier()
        pltpu.sync_copy(acc.at[pl.ds(s * ZROWS, ZROWS)],
                        out_hbm.at[c, pl.ds(s * ZROWS, ZROWS)])

    return functools.partial(
        pl.kernel,
        out_type=jax.ShapeDtypeStruct((2, NACC, D), _f32),
        mesh=mesh,
        scratch_types=[
            pltpu.VMEM((nchunks * CH,), jnp.int32),
            pltpu.VMEM((2, CH, D), _f32),
            pltpu.VMEM_SHARED((NACC, D), _f32),
            pltpu.SemaphoreType.DMA,
            pltpu.SemaphoreType.DMA,
            pltpu.SemaphoreType.DMA,
            pltpu.SemaphoreType.DMA,
        ],
    )(body)


def _node_body(n_ref, am_ref, aw_ref, w_ref, b1_ref, w2_ref, b2_ref, o_ref):
    n = n_ref[...]
    am = am_ref[0] + am_ref[1]
    aw = aw_ref[0] + aw_ref[1]
    pre = (jnp.dot(n, w_ref[0], preferred_element_type=_f32)
           + jnp.dot(am, w_ref[1], preferred_element_type=_f32)
           + jnp.dot(aw, w_ref[2], preferred_element_type=_f32)
           + b1_ref[...])
    h = jnp.maximum(pre, 0.0)
    o_ref[...] = jnp.dot(h, w2_ref[...], preferred_element_type=_f32) \
        + b2_ref[...] + n


def _node_mlp(node, am_p, aw_p, nws, b1, w2, b2):
    blk = 1000
    return pl.pallas_call(
        _node_body,
        grid=(N // blk,),
        in_specs=[
            pl.BlockSpec((blk, D), lambda i: (i, 0)),
            pl.BlockSpec((2, blk, D), lambda i: (0, i, 0)),
            pl.BlockSpec((2, blk, D), lambda i: (0, i, 0)),
            pl.BlockSpec((3, D, D), lambda i: (0, 0, 0)),
            pl.BlockSpec((1, D), lambda i: (0, 0)),
            pl.BlockSpec((D, D), lambda i: (0, 0)),
            pl.BlockSpec((1, D), lambda i: (0, 0)),
        ],
        out_specs=pl.BlockSpec((blk, D), lambda i: (i, 0)),
        out_shape=jax.ShapeDtypeStruct((N, D), _f32),
    )(node, am_p, aw_p, nws, b1, w2, b2)


def kernel(node_features, mesh_edge_features, world_edge_features,
           mesh_senders, mesh_receivers, world_senders, world_receivers,
           mesh_W1, mesh_b1, mesh_W2, mesh_b2,
           world_W1, world_b1, world_W2, world_b2,
           node_W1, node_b1, node_W2, node_b2):
    pm = EPM - E_MESH
    pw = EPW - E_WORLD
    gpad_m = jnp.arange(pm, dtype=jnp.int32) % N
    gpad_w = jnp.arange(pw, dtype=jnp.int32) % N
    ism = jnp.concatenate([mesh_senders, gpad_m])
    irm = jnp.concatenate([mesh_receivers, gpad_m])
    isw = jnp.concatenate([world_senders, gpad_w])
    irw = jnp.concatenate([world_receivers, gpad_w])
    spad_m = N + jnp.arange(pm, dtype=jnp.int32) % (NACC - N)
    spad_w = N + jnp.arange(pw, dtype=jnp.int32) % (NACC - N)
    srm = jnp.concatenate([mesh_receivers, spad_m])
    srw = jnp.concatenate([world_receivers, spad_w])
    efm = jnp.pad(mesh_edge_features, ((0, pm), (0, 0)))
    efw = jnp.pad(world_edge_features, ((0, pw), (0, 0)))
    zeros = jnp.zeros((ZROWS, D), _f32)

    ws_proj = jnp.stack([mesh_W1[:D], mesh_W1[D:2 * D],
                         world_W1[:D], world_W1[D:2 * D]])
    nws = jnp.stack([node_W1[:D], node_W1[D:2 * D], node_W1[2 * D:]])

    t = _project(node_features, ws_proj)
    gsm, grm = _get_sc_gather(CPM, EPM, 0)(t, ism, irm)
    gsw, grw = _get_sc_gather(CPW, EPW, 2)(t, isw, irw)
    new_m, out_m = _edge_mlp(gsm, grm, efm, mesh_W1[2 * D:],
                             mesh_b1.reshape(1, D),
                             mesh_W2, mesh_b2.reshape(1, D), E_MESH)
    am_p = _get_sc_scatter(CPM)(new_m, srm, zeros)
    new_w, out_w = _edge_mlp(gsw, grw, efw, world_W1[2 * D:],
                             world_b1.reshape(1, D),
                             world_W2, world_b2.reshape(1, D), E_WORLD)
    aw_p = _get_sc_scatter(CPW)(new_w, srw, zeros)
    out_n = _node_mlp(node_features, am_p, aw_p, nws,
                      node_b1.reshape(1, D), node_W2, node_b2.reshape(1, D))
    return (out_n, out_m, out_w)


# speedup vs baseline: 4.2251x; 1.0119x over previous
import functools

import jax
import jax.numpy as jnp
from jax import lax
from jax.experimental import pallas as pl
from jax.experimental.pallas import tpu as pltpu
from jax.experimental.pallas import tpu_sc as plsc

N = 10000
D = 128
E_MESH = 320000
E_WORLD = 80000
CH = 128
NTILES = 32
EPM = 327680
EPW = 81920
CPM = EPM // (NTILES * CH)
CPW = EPW // (NTILES * CH)
NACC = 10240
ZROWS = NACC // 16

_f32 = jnp.float32


def _proj_body(n_ref, w_ref, t_ref):
    t_ref[0] = jnp.dot(n_ref[...], w_ref[0], preferred_element_type=_f32)


def _project(node, ws):
    blk = 1000
    return pl.pallas_call(
        _proj_body,
        grid=(4, N // blk),
        in_specs=[
            pl.BlockSpec((blk, D), lambda j, i: (i, 0)),
            pl.BlockSpec((1, D, D), lambda j, i: (j, 0, 0)),
        ],
        out_specs=pl.BlockSpec((1, blk, D), lambda j, i: (j, i, 0)),
        out_shape=jax.ShapeDtypeStruct((4, NACC, D), _f32),
    )(node, ws)


@functools.cache
def _get_sc_gather(nchunks, ep, p_base):
    mesh = plsc.VectorSubcoreMesh(
        core_axis_name="c", subcore_axis_name="s",
        num_cores=2, num_subcores=16)

    def body(t_hbm, is_hbm, ir_hbm, gs_hbm, gr_hbm,
             idx_v, buf_v, tab_sh, sg0, sg1, so0, so1):
        wid = lax.axis_index("s") * 2 + lax.axis_index("c")
        s = lax.axis_index("s")
        sgs = (sg0, sg1)
        sos = (so0, so1)

        def phase(p, i_hbm, dst_hbm):
            pltpu.sync_copy(t_hbm.at[p, pl.ds(s * ZROWS, ZROWS)],
                            tab_sh.at[pl.ds(s * ZROWS, ZROWS)])
            n_idx = nchunks * CH
            pltpu.sync_copy(i_hbm.at[pl.ds(wid * n_idx, n_idx)], idx_v)
            plsc.subcore_barrier()

            def issue(k, b):
                @pl.when(k >= 2)
                def _():
                    pltpu.make_async_copy(
                        buf_v.at[b],
                        dst_hbm.at[pl.ds((wid * nchunks + k - 2) * CH, CH)],
                        sos[b]).wait()
                pltpu.async_copy(tab_sh.at[idx_v.at[pl.ds(k * CH, CH)]],
                                 buf_v.at[b], sgs[b])

            def process(j, bp):
                pltpu.make_async_copy(
                    tab_sh.at[idx_v.at[pl.ds(j * CH, CH)]],
                    buf_v.at[bp], sgs[bp]).wait()
                pltpu.async_copy(
                    buf_v.at[bp],
                    dst_hbm.at[pl.ds((wid * nchunks + j) * CH, CH)], sos[bp])

            def outer(g, _):
                for b in range(2):
                    k = 2 * g + b
                    issue(k, b)
                    j = k - 1
                    bp = (b - 1) % 2

                    @pl.when(j >= 0)
                    def _():
                        process(j, bp)
                return 0

            lax.fori_loop(0, nchunks // 2, outer, 0)
            process(nchunks - 1, (nchunks - 1) % 2)
            for b in range(2):
                pltpu.make_async_copy(
                    buf_v.at[b],
                    dst_hbm.at[
                        pl.ds((wid * nchunks + nchunks - 2 + b) * CH, CH)],
                    sos[b]).wait()
            plsc.subcore_barrier()

        phase(p_base, is_hbm, gs_hbm)
        phase(p_base + 1, ir_hbm, gr_hbm)

    return functools.partial(
        pl.kernel,
        out_type=[jax.ShapeDtypeStruct((ep, D), _f32),
                  jax.ShapeDtypeStruct((ep, D), _f32)],
        mesh=mesh,
        scratch_types=[
            pltpu.VMEM((nchunks * CH,), jnp.int32),
            pltpu.VMEM((2, CH, D), _f32),
            pltpu.VMEM_SHARED((NACC, D), _f32),
            pltpu.SemaphoreType.DMA,
            pltpu.SemaphoreType.DMA,
            pltpu.SemaphoreType.DMA,
            pltpu.SemaphoreType.DMA,
        ],
    )(body)


def _edge_body(gs_ref, gr_ref, e_ref, w1c_ref, b1_ref, w2_ref, b2_ref,
               new_ref, out_ref):
    e = e_ref[...]
    pre = (gs_ref[...] + gr_ref[...]
           + jnp.dot(e, w1c_ref[...], preferred_element_type=_f32)
           + b1_ref[...])
    h = jnp.maximum(pre, 0.0)
    new = jnp.dot(h, w2_ref[...], preferred_element_type=_f32) + b2_ref[...]
    new_ref[...] = new
    out_ref[...] = new + e


def _edge_mlp(gs, gr, ef, w1c, b1, w2, b2, e_real):
    ep = gs.shape[0]
    blk = 2048
    grid = (e_real + blk - 1) // blk
    return pl.pallas_call(
        _edge_body,
        grid=(grid,),
        in_specs=[
            pl.BlockSpec((blk, D), lambda i: (i, 0)),
            pl.BlockSpec((blk, D), lambda i: (i, 0)),
            pl.BlockSpec((blk, D), lambda i: (i, 0)),
            pl.BlockSpec((D, D), lambda i: (0, 0)),
            pl.BlockSpec((1, D), lambda i: (0, 0)),
            pl.BlockSpec((D, D), lambda i: (0, 0)),
            pl.BlockSpec((1, D), lambda i: (0, 0)),
        ],
        out_specs=[
            pl.BlockSpec((blk, D), lambda i: (i, 0)),
            pl.BlockSpec((blk, D), lambda i: (i, 0)),
        ],
        out_shape=[jax.ShapeDtypeStruct((ep, D), _f32),
                   jax.ShapeDtypeStruct((e_real, D), _f32)],
    )(gs, gr, ef, w1c, b1, w2, b2)


@functools.cache
def _get_sc_scatter(nchunks):
    mesh = plsc.VectorSubcoreMesh(
        core_axis_name="c", subcore_axis_name="s",
        num_cores=2, num_subcores=16)

    def body(src_hbm, r_hbm, z_hbm, out_hbm, idx_v, buf_v, acc,
             si0, si1, sa0, sa1):
        c = lax.axis_index("c")
        s = lax.axis_index("s")
        wid = s * 2 + c
        sis = (si0, si1)
        sas = (sa0, sa1)

        pltpu.sync_copy(z_hbm, acc.at[pl.ds(s * ZROWS, ZROWS)])
        n_idx = nchunks * CH
        pltpu.sync_copy(r_hbm.at[pl.ds(wid * n_idx, n_idx)], idx_v)
        plsc.subcore_barrier()

        def issue(k, b):
            @pl.when(k >= 2)
            def _():
                pltpu.make_async_copy(
                    buf_v.at[b],
                    acc.at[idx_v.at[pl.ds((k - 2) * CH, CH)]],
                    sas[b]).wait()
            pltpu.async_copy(
                src_hbm.at[pl.ds((wid * nchunks + k) * CH, CH)],
                buf_v.at[b], sis[b])

        def process(j, bp):
            pltpu.make_async_copy(
                src_hbm.at[pl.ds((wid * nchunks + j) * CH, CH)],
                buf_v.at[bp], sis[bp]).wait()
            pltpu.async_copy(buf_v.at[bp],
                             acc.at[idx_v.at[pl.ds(j * CH, CH)]],
                             sas[bp], add=True)

        def outer(g, _):
            for b in range(2):
                k = 2 * g + b
                issue(k, b)
                j = k - 1
                bp = (b - 1) % 2

                @pl.when(j >= 0)
                def _():
                    process(j, bp)
            return 0

        lax.fori_loop(0, nchunks // 2, outer, 0)
        process(nchunks - 1, (nchunks - 1) % 2)
        for b in range(2):
            pltpu.make_async_copy(
                buf_v.at[b],
                acc.at[idx_v.at[pl.ds((nchunks - 2 + b) * CH, CH)]],
                sas[b]).wait()
        plsc.subcore_barrier()
        pltpu.sync_copy(acc.at[pl.ds(s * ZROWS, ZROWS)],
                        out_hbm.at[c, pl.ds(s * ZROWS, ZROWS)])

    return functools.partial(
        pl.kernel,
        out_type=jax.ShapeDtypeStruct((2, NACC, D), _f32),
        mesh=mesh,
        scratch_types=[
            pltpu.VMEM((nchunks * CH,), jnp.int32),
            pltpu.VMEM((2, CH, D), _f32),
            pltpu.VMEM_SHARED((NACC, D), _f32),
            pltpu.SemaphoreType.DMA,
            pltpu.SemaphoreType.DMA,
            pltpu.SemaphoreType.DMA,
            pltpu.SemaphoreType.DMA,
        ],
    )(body)


def _node_body(n_ref, am_ref, aw_ref, w_ref, b1_ref, w2_ref, b2_ref, o_ref):
    n = n_ref[...]
    am = am_ref[0] + am_ref[1]
    aw = aw_ref[0] + aw_ref[1]
    pre = (jnp.dot(n, w_ref[0], preferred_element_type=_f32)
           + jnp.dot(am, w_ref[1], preferred_element_type=_f32)
           + jnp.dot(aw, w_ref[2], preferred_element_type=_f32)
           + b1_ref[...])
    h = jnp.maximum(pre, 0.0)
    o_ref[...] = jnp.dot(h, w2_ref[...], preferred_element_type=_f32) \
        + b2_ref[...] + n


def _node_mlp(node, am_p, aw_p, nws, b1, w2, b2):
    blk = 1000
    return pl.pallas_call(
        _node_body,
        grid=(N // blk,),
        in_specs=[
            pl.BlockSpec((blk, D), lambda i: (i, 0)),
            pl.BlockSpec((2, blk, D), lambda i: (0, i, 0)),
            pl.BlockSpec((2, blk, D), lambda i: (0, i, 0)),
            pl.BlockSpec((3, D, D), lambda i: (0, 0, 0)),
            pl.BlockSpec((1, D), lambda i: (0, 0)),
            pl.BlockSpec((D, D), lambda i: (0, 0)),
            pl.BlockSpec((1, D), lambda i: (0, 0)),
        ],
        out_specs=pl.BlockSpec((blk, D), lambda i: (i, 0)),
        out_shape=jax.ShapeDtypeStruct((N, D), _f32),
    )(node, am_p, aw_p, nws, b1, w2, b2)


def kernel(node_features, mesh_edge_features, world_edge_features,
           mesh_senders, mesh_receivers, world_senders, world_receivers,
           mesh_W1, mesh_b1, mesh_W2, mesh_b2,
           world_W1, world_b1, world_W2, world_b2,
           node_W1, node_b1, node_W2, node_b2):
    pm = EPM - E_MESH
    pw = EPW - E_WORLD
    gpad_m = jnp.arange(pm, dtype=jnp.int32) % N
    gpad_w = jnp.arange(pw, dtype=jnp.int32) % N
    ism = jnp.concatenate([mesh_senders, gpad_m])
    irm = jnp.concatenate([mesh_receivers, gpad_m])
    isw = jnp.concatenate([world_senders, gpad_w])
    irw = jnp.concatenate([world_receivers, gpad_w])
    spad_m = N + jnp.arange(pm, dtype=jnp.int32) % (NACC - N)
    spad_w = N + jnp.arange(pw, dtype=jnp.int32) % (NACC - N)
    srm = jnp.concatenate([mesh_receivers, spad_m])
    srw = jnp.concatenate([world_receivers, spad_w])
    efm = jnp.pad(mesh_edge_features, ((0, pm), (0, 0)))
    efw = jnp.pad(world_edge_features, ((0, pw), (0, 0)))
    zeros = jnp.zeros((ZROWS, D), _f32)

    ws_proj = jnp.stack([mesh_W1[:D], mesh_W1[D:2 * D],
                         world_W1[:D], world_W1[D:2 * D]])
    nws = jnp.stack([node_W1[:D], node_W1[D:2 * D], node_W1[2 * D:]])

    t = _project(node_features, ws_proj)
    gsm, grm = _get_sc_gather(CPM, EPM, 0)(t, ism, irm)
    gsw, grw = _get_sc_gather(CPW, EPW, 2)(t, isw, irw)
    new_m, out_m = _edge_mlp(gsm, grm, efm, mesh_W1[2 * D:],
                             mesh_b1.reshape(1, D),
                             mesh_W2, mesh_b2.reshape(1, D), E_MESH)
    am_p = _get_sc_scatter(CPM)(new_m, srm, zeros)
    new_w, out_w = _edge_mlp(gsw, grw, efw, world_W1[2 * D:],
                             world_b1.reshape(1, D),
                             world_W2, world_b2.reshape(1, D), E_WORLD)
    aw_p = _get_sc_scatter(CPW)(new_w, srw, zeros)
    out_n = _node_mlp(node_features, am_p, aw_p, nws,
                      node_b1.reshape(1, D), node_W2, node_b2.reshape(1, D))
    return (out_n, out_m, out_w)


# speedup vs baseline: 4.2314x; 1.0015x over previous
import functools

import jax
import jax.numpy as jnp
from jax import lax
from jax.experimental import pallas as pl
from jax.experimental.pallas import tpu as pltpu
from jax.experimental.pallas import tpu_sc as plsc

N = 10000
D = 128
E_MESH = 320000
E_WORLD = 80000
CH = 128
NTILES = 32
EPM = 327680
EPW = 81920
CPM = EPM // (NTILES * CH)
CPW = EPW // (NTILES * CH)
NACC = 10240
ZROWS = NACC // 16

_f32 = jnp.float32


def _proj_body(n_ref, w_ref, t_ref):
    t_ref[0] = jnp.dot(n_ref[...], w_ref[0], preferred_element_type=_f32)


def _project(node, ws):
    blk = 1000
    return pl.pallas_call(
        _proj_body,
        grid=(4, N // blk),
        in_specs=[
            pl.BlockSpec((blk, D), lambda j, i: (i, 0)),
            pl.BlockSpec((1, D, D), lambda j, i: (j, 0, 0)),
        ],
        out_specs=pl.BlockSpec((1, blk, D), lambda j, i: (j, i, 0)),
        out_shape=jax.ShapeDtypeStruct((4, NACC, D), _f32),
    )(node, ws)


@functools.cache
def _get_sc_gather(nchunks, ep, p_base):
    mesh = plsc.VectorSubcoreMesh(
        core_axis_name="c", subcore_axis_name="s",
        num_cores=2, num_subcores=16)

    def body(t_hbm, is_hbm, ir_hbm, gs_hbm, gr_hbm,
             idx_v, buf_v, tab_sh, sg0, sg1, so0, so1):
        wid = lax.axis_index("s") * 2 + lax.axis_index("c")
        s = lax.axis_index("s")
        sgs = (sg0, sg1)
        sos = (so0, so1)

        def phase(p, i_hbm, dst_hbm):
            pltpu.sync_copy(t_hbm.at[p, pl.ds(s * ZROWS, ZROWS)],
                            tab_sh.at[pl.ds(s * ZROWS, ZROWS)])
            n_idx = nchunks * CH
            pltpu.sync_copy(i_hbm.at[pl.ds(wid * n_idx, n_idx)], idx_v)
            plsc.subcore_barrier()

            def issue(k, b):
                @pl.when(k >= 2)
                def _():
                    pltpu.make_async_copy(
                        buf_v.at[b],
                        dst_hbm.at[pl.ds((wid * nchunks + k - 2) * CH, CH)],
                        sos[b]).wait()
                pltpu.async_copy(tab_sh.at[idx_v.at[pl.ds(k * CH, CH)]],
                                 buf_v.at[b], sgs[b])

            def process(j, bp):
                pltpu.make_async_copy(
                    tab_sh.at[idx_v.at[pl.ds(j * CH, CH)]],
                    buf_v.at[bp], sgs[bp]).wait()
                pltpu.async_copy(
                    buf_v.at[bp],
                    dst_hbm.at[pl.ds((wid * nchunks + j) * CH, CH)], sos[bp])

            def outer(g, _):
                for b in range(2):
                    k = 2 * g + b
                    issue(k, b)
                    j = k - 1
                    bp = (b - 1) % 2

                    @pl.when(j >= 0)
                    def _():
                        process(j, bp)
                return 0

            lax.fori_loop(0, nchunks // 2, outer, 0)
            process(nchunks - 1, (nchunks - 1) % 2)
            for b in range(2):
                pltpu.make_async_copy(
                    buf_v.at[b],
                    dst_hbm.at[
                        pl.ds((wid * nchunks + nchunks - 2 + b) * CH, CH)],
                    sos[b]).wait()
            plsc.subcore_barrier()

        phase(p_base, is_hbm, gs_hbm)
        phase(p_base + 1, ir_hbm, gr_hbm)

    return functools.partial(
        pl.kernel,
        out_type=[jax.ShapeDtypeStruct((ep, D), _f32),
                  jax.ShapeDtypeStruct((ep, D), _f32)],
        mesh=mesh,
        scratch_types=[
            pltpu.VMEM((nchunks * CH,), jnp.int32),
            pltpu.VMEM((2, CH, D), _f32),
            pltpu.VMEM_SHARED((NACC, D), _f32),
            pltpu.SemaphoreType.DMA,
            pltpu.SemaphoreType.DMA,
            pltpu.SemaphoreType.DMA,
            pltpu.SemaphoreType.DMA,
        ],
    )(body)


def _edge_body(gs_ref, gr_ref, e_ref, w1c_ref, b1_ref, w2_ref, b2_ref,
               new_ref, out_ref):
    e = e_ref[...]
    pre = (gs_ref[...] + gr_ref[...]
           + jnp.dot(e, w1c_ref[...], preferred_element_type=_f32)
           + b1_ref[...])
    h = jnp.maximum(pre, 0.0)
    new = jnp.dot(h, w2_ref[...], preferred_element_type=_f32) + b2_ref[...]
    new_ref[...] = new
    out_ref[...] = new + e


def _edge_mlp(gs, gr, ef, w1c, b1, w2, b2, e_real):
    ep = gs.shape[0]
    blk = 2048
    grid = (e_real + blk - 1) // blk
    return pl.pallas_call(
        _edge_body,
        grid=(grid,),
        in_specs=[
            pl.BlockSpec((blk, D), lambda i: (i, 0)),
            pl.BlockSpec((blk, D), lambda i: (i, 0)),
            pl.BlockSpec((blk, D), lambda i: (i, 0)),
            pl.BlockSpec((D, D), lambda i: (0, 0)),
            pl.BlockSpec((1, D), lambda i: (0, 0)),
            pl.BlockSpec((D, D), lambda i: (0, 0)),
            pl.BlockSpec((1, D), lambda i: (0, 0)),
        ],
        out_specs=[
            pl.BlockSpec((blk, D), lambda i: (i, 0)),
            pl.BlockSpec((blk, D), lambda i: (i, 0)),
        ],
        out_shape=[jax.ShapeDtypeStruct((ep, D), _f32),
                   jax.ShapeDtypeStruct((e_real, D), _f32)],
    )(gs, gr, ef, w1c, b1, w2, b2)


@functools.cache
def _get_sc_scatter(nchunks):
    mesh = plsc.VectorSubcoreMesh(
        core_axis_name="c", subcore_axis_name="s",
        num_cores=2, num_subcores=16)

    def body(src_hbm, r_hbm, z_hbm, out_hbm, idx_v, buf_v, acc,
             si0, si1, sa0, sa1):
        c = lax.axis_index("c")
        s = lax.axis_index("s")
        wid = s * 2 + c
        sis = (si0, si1)
        sas = (sa0, sa1)

        pltpu.sync_copy(z_hbm.at[pl.ds(s * ZROWS, ZROWS)],
                        acc.at[pl.ds(s * ZROWS, ZROWS)])
        n_idx = nchunks * CH
        pltpu.sync_copy(r_hbm.at[pl.ds(wid * n_idx, n_idx)], idx_v)
        plsc.subcore_barrier()

        def issue(k, b):
            @pl.when(k >= 2)
            def _():
                pltpu.make_async_copy(
                    buf_v.at[b],
                    acc.at[idx_v.at[pl.ds((k - 2) * CH, CH)]],
                    sas[b]).wait()
            pltpu.async_copy(
                src_hbm.at[pl.ds((wid * nchunks + k) * CH, CH)],
                buf_v.at[b], sis[b])

        def process(j, bp):
            pltpu.make_async_copy(
                src_hbm.at[pl.ds((wid * nchunks + j) * CH, CH)],
                buf_v.at[bp], sis[bp]).wait()
            pltpu.async_copy(buf_v.at[bp],
                             acc.at[idx_v.at[pl.ds(j * CH, CH)]],
                             sas[bp], add=True)

        def outer(g, _):
            for b in range(2):
                k = 2 * g + b
                issue(k, b)
                j = k - 1
                bp = (b - 1) % 2

                @pl.when(j >= 0)
                def _():
                    process(j, bp)
            return 0

        lax.fori_loop(0, nchunks // 2, outer, 0)
        process(nchunks - 1, (nchunks - 1) % 2)
        for b in range(2):
            pltpu.make_async_copy(
                buf_v.at[b],
                acc.at[idx_v.at[pl.ds((nchunks - 2 + b) * CH, CH)]],
                sas[b]).wait()
        plsc.subcore_barrier()
        pltpu.sync_copy(acc.at[pl.ds(s * ZROWS, ZROWS)],
                        out_hbm.at[c, pl.ds(s * ZROWS, ZROWS)])

    return functools.partial(
        pl.kernel,
        out_type=jax.ShapeDtypeStruct((2, NACC, D), _f32),
        mesh=mesh,
        scratch_types=[
            pltpu.VMEM((nchunks * CH,), jnp.int32),
            pltpu.VMEM((2, CH, D), _f32),
            pltpu.VMEM_SHARED((NACC, D), _f32),
            pltpu.SemaphoreType.DMA,
            pltpu.SemaphoreType.DMA,
            pltpu.SemaphoreType.DMA,
            pltpu.SemaphoreType.DMA,
        ],
    )(body)


def _node_body(n_ref, am_ref, aw_ref, w_ref, b1_ref, w2_ref, b2_ref, o_ref):
    n = n_ref[...]
    am = am_ref[0] + am_ref[1]
    aw = aw_ref[0] + aw_ref[1]
    pre = (jnp.dot(n, w_ref[0], preferred_element_type=_f32)
           + jnp.dot(am, w_ref[1], preferred_element_type=_f32)
           + jnp.dot(aw, w_ref[2], preferred_element_type=_f32)
           + b1_ref[...])
    h = jnp.maximum(pre, 0.0)
    o_ref[...] = jnp.dot(h, w2_ref[...], preferred_element_type=_f32) \
        + b2_ref[...] + n


def _node_mlp(node, am_p, aw_p, nws, b1, w2, b2):
    blk = 1000
    return pl.pallas_call(
        _node_body,
        grid=(N // blk,),
        in_specs=[
            pl.BlockSpec((blk, D), lambda i: (i, 0)),
            pl.BlockSpec((2, blk, D), lambda i: (0, i, 0)),
            pl.BlockSpec((2, blk, D), lambda i: (0, i, 0)),
            pl.BlockSpec((3, D, D), lambda i: (0, 0, 0)),
            pl.BlockSpec((1, D), lambda i: (0, 0)),
            pl.BlockSpec((D, D), lambda i: (0, 0)),
            pl.BlockSpec((1, D), lambda i: (0, 0)),
        ],
        out_specs=pl.BlockSpec((blk, D), lambda i: (i, 0)),
        out_shape=jax.ShapeDtypeStruct((N, D), _f32),
    )(node, am_p, aw_p, nws, b1, w2, b2)


def kernel(node_features, mesh_edge_features, world_edge_features,
           mesh_senders, mesh_receivers, world_senders, world_receivers,
           mesh_W1, mesh_b1, mesh_W2, mesh_b2,
           world_W1, world_b1, world_W2, world_b2,
           node_W1, node_b1, node_W2, node_b2):
    pm = EPM - E_MESH
    pw = EPW - E_WORLD
    gpad_m = jnp.arange(pm, dtype=jnp.int32) % N
    gpad_w = jnp.arange(pw, dtype=jnp.int32) % N
    ism = jnp.concatenate([mesh_senders, gpad_m])
    irm = jnp.concatenate([mesh_receivers, gpad_m])
    isw = jnp.concatenate([world_senders, gpad_w])
    irw = jnp.concatenate([world_receivers, gpad_w])
    spad_m = N + jnp.arange(pm, dtype=jnp.int32) % (NACC - N)
    spad_w = N + jnp.arange(pw, dtype=jnp.int32) % (NACC - N)
    srm = jnp.concatenate([mesh_receivers, spad_m])
    srw = jnp.concatenate([world_receivers, spad_w])
    efm = jnp.pad(mesh_edge_features, ((0, pm), (0, 0)))
    efw = jnp.pad(world_edge_features, ((0, pw), (0, 0)))
    zeros = jnp.zeros((NACC, D), _f32)

    ws_proj = jnp.stack([mesh_W1[:D], mesh_W1[D:2 * D],
                         world_W1[:D], world_W1[D:2 * D]])
    nws = jnp.stack([node_W1[:D], node_W1[D:2 * D], node_W1[2 * D:]])

    t = _project(node_features, ws_proj)
    gsw, grw = _get_sc_gather(CPW, EPW, 2)(t, isw, irw)
    gsm, grm = _get_sc_gather(CPM, EPM, 0)(t, ism, irm)
    new_w, out_w = _edge_mlp(gsw, grw, efw, world_W1[2 * D:],
                             world_b1.reshape(1, D),
                             world_W2, world_b2.reshape(1, D), E_WORLD)
    aw_p = _get_sc_scatter(CPW)(new_w, srw, zeros)
    new_m, out_m = _edge_mlp(gsm, grm, efm, mesh_W1[2 * D:],
                             mesh_b1.reshape(1, D),
                             mesh_W2, mesh_b2.reshape(1, D), E_MESH)
    am_p = _get_sc_scatter(CPM)(new_m, srm, zeros)
    out_n = _node_mlp(node_features, am_p, aw_p, nws,
                      node_b1.reshape(1, D), node_W2, node_b2.reshape(1, D))
    return (out_n, out_m, out_w)
